# baseline (speedup 1.0000x reference)
import functools

import jax
import jax.numpy as jnp
from jax import lax
from jax.experimental import pallas as pl
from jax.experimental.pallas import tpu as pltpu
from jax.experimental.pallas import tpu_sc as plsc

_B, _CIN, _COUT, _D = 64, 256, 128, 1024
_NC, _NS, _L = 2, 16, 16
_NW = _NC * _NS
_R = _B * _COUT
_RPW = _R // _NW
_BPW = _B // _NW
_CH = 8
_NCHUNK = _RPW // _CH
_NBUF = 14

_mesh = plsc.VectorSubcoreMesh(core_axis_name="c", subcore_axis_name="s")


@functools.partial(
    pl.kernel,
    mesh=_mesh,
    out_type=jax.ShapeDtypeStruct((_R, _D), jnp.float32),
    scratch_types=[
        pltpu.VMEM((_RPW,), jnp.int32),
        pltpu.VMEM((_NBUF * _CH, _D), jnp.float32),
        pltpu.SemaphoreType.DMA((_NBUF,)),
        pltpu.SemaphoreType.DMA((_NBUF,)),
    ],
)
def _gather(x_hbm, idx_hbm, out_hbm, rows_idx_v, ring_v, gsem, ssem):
    wid = lax.axis_index("s") * _NC + lax.axis_index("c")
    base = wid * _RPW
    pltpu.sync_copy(idx_hbm, rows_idx_v.at[pl.ds(0, _COUT)])
    b_hi = (wid * _BPW + 1) * _CIN
    b_lo = (wid * _BPW) * _CIN
    for k in range(_COUT // _L):
        j0 = k * _L
        rows_idx_v[pl.ds(_COUT + j0, _L)] = rows_idx_v[pl.ds(j0, _L)] + b_hi
    for k in range(_COUT // _L):
        j0 = k * _L
        rows_idx_v[pl.ds(j0, _L)] = rows_idx_v[pl.ds(j0, _L)] + b_lo

    def _slot(ci):
        return lax.rem(ci, _NBUF)

    def _gather_start(ci):
        p = _slot(ci)
        pltpu.async_copy(
            x_hbm.at[rows_idx_v.at[pl.ds(ci * _CH, _CH)]],
            ring_v.at[pl.ds(p * _CH, _CH)],
            gsem.at[p],
        )

    def _scatter_start(ci):
        p = _slot(ci)
        pltpu.async_copy(
            ring_v.at[pl.ds(p * _CH, _CH)],
            out_hbm.at[pl.ds(base + ci * _CH, _CH)],
            ssem.at[p],
        )

    def _gather_wait(ci):
        p = _slot(ci)
        pltpu.make_async_copy(
            x_hbm.at[rows_idx_v.at[pl.ds(ci * _CH, _CH)]],
            ring_v.at[pl.ds(p * _CH, _CH)],
            gsem.at[p],
        ).wait()

    def _scatter_wait(ci):
        p = _slot(ci)
        pltpu.make_async_copy(
            ring_v.at[pl.ds(p * _CH, _CH)],
            out_hbm.at[pl.ds(base + ci * _CH, _CH)],
            ssem.at[p],
        ).wait()

    for ci in range(_NBUF):
        _gather_start(ci)

    def body(ci, carry):
        _gather_wait(ci)
        _scatter_start(ci)

        @pl.when(ci + _NBUF < _NCHUNK)
        def _():
            _scatter_wait(ci)
            _gather_start(ci + _NBUF)

        return carry

    lax.fori_loop(0, _NCHUNK, body, 0)
    for ci in range(_NCHUNK - _NBUF, _NCHUNK):
        _scatter_wait(ci)


def kernel(x, indices):
    out = _gather(x.reshape(_B * _CIN, _D), indices)
    return out.reshape(_B, _COUT, _D)


# speedup vs baseline: 1.0132x; 1.0132x over previous
import functools

import jax
import jax.numpy as jnp
from jax import lax
from jax.experimental import pallas as pl
from jax.experimental.pallas import tpu as pltpu
from jax.experimental.pallas import tpu_sc as plsc

_B, _CIN, _COUT, _D = 64, 256, 128, 1024
_NC, _NS, _L = 2, 16, 16
_NW = _NC * _NS
_R = _B * _COUT
_RPW = _R // _NW
_BPW = _B // _NW
_CH = 16
_NCHUNK = _RPW // _CH
_NBUF = 7

_mesh = plsc.VectorSubcoreMesh(core_axis_name="c", subcore_axis_name="s")


@functools.partial(
    pl.kernel,
    mesh=_mesh,
    out_type=jax.ShapeDtypeStruct((_R, _D), jnp.float32),
    scratch_types=[
        pltpu.VMEM((_RPW,), jnp.int32),
        pltpu.VMEM((_NBUF * _CH, _D), jnp.float32),
        pltpu.SemaphoreType.DMA((_NBUF,)),
        pltpu.SemaphoreType.DMA((_NBUF,)),
    ],
)
def _gather(x_hbm, idx_hbm, out_hbm, rows_idx_v, ring_v, gsem, ssem):
    wid = lax.axis_index("s") * _NC + lax.axis_index("c")
    base = wid * _RPW
    pltpu.sync_copy(idx_hbm, rows_idx_v.at[pl.ds(0, _COUT)])
    b_hi = (wid * _BPW + 1) * _CIN
    b_lo = (wid * _BPW) * _CIN
    for k in range(_COUT // _L):
        j0 = k * _L
        rows_idx_v[pl.ds(_COUT + j0, _L)] = rows_idx_v[pl.ds(j0, _L)] + b_hi
    for k in range(_COUT // _L):
        j0 = k * _L
        rows_idx_v[pl.ds(j0, _L)] = rows_idx_v[pl.ds(j0, _L)] + b_lo

    def _slot(ci):
        return lax.rem(ci, _NBUF)

    def _gather_start(ci):
        p = _slot(ci)
        pltpu.async_copy(
            x_hbm.at[rows_idx_v.at[pl.ds(ci * _CH, _CH)]],
            ring_v.at[pl.ds(p * _CH, _CH)],
            gsem.at[p],
        )

    def _scatter_start(ci):
        p = _slot(ci)
        pltpu.async_copy(
            ring_v.at[pl.ds(p * _CH, _CH)],
            out_hbm.at[pl.ds(base + ci * _CH, _CH)],
            ssem.at[p],
        )

    def _gather_wait(ci):
        p = _slot(ci)
        pltpu.make_async_copy(
            x_hbm.at[rows_idx_v.at[pl.ds(ci * _CH, _CH)]],
            ring_v.at[pl.ds(p * _CH, _CH)],
            gsem.at[p],
        ).wait()

    def _scatter_wait(ci):
        p = _slot(ci)
        pltpu.make_async_copy(
            ring_v.at[pl.ds(p * _CH, _CH)],
            out_hbm.at[pl.ds(base + ci * _CH, _CH)],
            ssem.at[p],
        ).wait()

    for ci in range(_NBUF):
        _gather_start(ci)

    def body(ci, carry):
        _gather_wait(ci)
        _scatter_start(ci)

        @pl.when(ci + _NBUF < _NCHUNK)
        def _():
            _scatter_wait(ci)
            _gather_start(ci + _NBUF)

        return carry

    lax.fori_loop(0, _NCHUNK, body, 0)
    for ci in range(_NCHUNK - _NBUF, _NCHUNK):
        _scatter_wait(ci)


def kernel(x, indices):
    out = _gather(x.reshape(_B * _CIN, _D), indices)
    return out.reshape(_B, _COUT, _D)


# speedup vs baseline: 1.0144x; 1.0011x over previous
import functools

import jax
import jax.numpy as jnp
from jax import lax
from jax.experimental import pallas as pl
from jax.experimental.pallas import tpu as pltpu
from jax.experimental.pallas import tpu_sc as plsc

_B, _CIN, _COUT, _D = 64, 256, 128, 1024
_NC, _NS, _L = 2, 16, 16
_NW = _NC * _NS
_BPW = _B // _NW
_CH = 16
_CPB = _COUT // _CH
_NCHUNK = _BPW * _CPB
_NBUF = 7

_mesh = plsc.VectorSubcoreMesh(core_axis_name="c", subcore_axis_name="s")


@functools.partial(
    pl.kernel,
    mesh=_mesh,
    out_type=jax.ShapeDtypeStruct((_B, _COUT, _D), jnp.float32),
    scratch_types=[
        pltpu.VMEM((_COUT,), jnp.int32),
        pltpu.VMEM((_NBUF * _CH, _D), jnp.float32),
        pltpu.SemaphoreType.DMA((_NBUF,)),
        pltpu.SemaphoreType.DMA((_NBUF,)),
    ],
)
def _gather(x_hbm, idx_hbm, out_hbm, tab_v, ring_v, gsem, ssem):
    wid = lax.axis_index("s") * _NC + lax.axis_index("c")
    b0 = wid * _BPW
    pltpu.sync_copy(idx_hbm, tab_v)

    def _slot(ci):
        return lax.rem(ci, _NBUF)

    def _bj(ci):
        bi = ci // _CPB
        return b0 + bi, (ci - bi * _CPB) * _CH

    def _gather_copy(ci):
        p = _slot(ci)
        b, j0 = _bj(ci)
        return pltpu.make_async_copy(
            x_hbm.at[b].at[tab_v.at[pl.ds(j0, _CH)]],
            ring_v.at[pl.ds(p * _CH, _CH)],
            gsem.at[p],
        )

    def _scatter_copy(ci):
        p = _slot(ci)
        b, j0 = _bj(ci)
        return pltpu.make_async_copy(
            ring_v.at[pl.ds(p * _CH, _CH)],
            out_hbm.at[b].at[pl.ds(j0, _CH)],
            ssem.at[p],
        )

    for ci in range(_NBUF):
        _gather_copy(ci).start()

    def body(ci, carry):
        _gather_copy(ci).wait()
        _scatter_copy(ci).start()

        @pl.when(ci + _NBUF < _NCHUNK)
        def _():
            _scatter_copy(ci).wait()
            _gather_copy(ci + _NBUF).start()

        return carry

    lax.fori_loop(0, _NCHUNK, body, 0)
    for ci in range(_NCHUNK - _NBUF, _NCHUNK):
        _scatter_copy(ci).wait()


def kernel(x, indices):
    return _gather(x, indices)
